# baseline (speedup 1.0000x reference)
import functools

import jax
import jax.numpy as jnp
from jax import lax
from jax.experimental import pallas as pl
from jax.experimental.pallas import tpu as pltpu
from jax.experimental.pallas import tpu_sc as plsc

_F32 = jnp.float32
_I32 = jnp.int32
_PARAMS = pltpu.CompilerParams(
    use_tc_tiling_on_sc=True,
    needs_layout_passes=False,
    disable_bounds_checks=True,
)


def _build_transpose(V, D, NC, NW):
    TCOLS = V // 128
    TAIL = V - TCOLS * 128
    mesh = plsc.VectorSubcoreMesh(core_axis_name="c", subcore_axis_name="s")

    base_cols = TCOLS // NW
    extra2 = (TCOLS - base_cols * NW) // 2

    PITCHO = 137

    @functools.partial(
        pl.kernel,
        mesh=mesh,
        out_type=jax.ShapeDtypeStruct((V, 2 * D), _F32),
        scratch_types=[
            pltpu.VMEM((D, 128), _F32),
            pltpu.VMEM((D, 128), _F32),
            pltpu.VMEM((128, PITCHO), _F32),
            pltpu.VMEM((128, PITCHO), _F32),
            *[pltpu.SemaphoreType.DMA for _ in range(4)],
        ],
        compiler_params=_PARAMS,
    )
    def ka(tt_hbm, tail_hbm, tpad_hbm, bin0, bin1, bout0, bout1,
           is0, is1, os0, os1):
        bins, bouts = (bin0, bin1), (bout0, bout1)
        isems, osems = (is0, is1), (os0, os1)
        wid = lax.axis_index("s") * NC + lax.axis_index("c")
        lo = wid * base_cols + 2 * jnp.minimum(wid, extra2)
        cnt = base_cols + jnp.where(wid < extra2, 2, 0)
        hi = lo + cnt

        rows16 = [lax.iota(_I32, 16) + 16 * m for m in range(8)]
        one = jnp.ones((16,), _I32)

        def in_start(c, par):
            pltpu.async_copy(
                tt_hbm.at[:, pl.ds(c * 128, 128)], bins[par], isems[par])

        def in_wait(par):
            pltpu.make_async_copy(
                tt_hbm.at[:, pl.ds(0, 128)], bins[par], isems[par]).wait()

        def out_start(c, par):
            pltpu.async_copy(
                bouts[par].at[:, pl.ds(0, 2 * D)],
                tpad_hbm.at[pl.ds(c * 128, 128), :], osems[par])

        def out_wait(par):
            pltpu.make_async_copy(
                bouts[par].at[:, pl.ds(0, 2 * D)],
                tpad_hbm.at[pl.ds(0, 128), :], osems[par]).wait()

        def transpose(par):
            @plsc.parallel_loop(0, D, unroll=16, carry=jnp.zeros((16,), _I32))
            def _f(f, fvec):
                for k in range(8):
                    vals = bins[par][f, pl.ds(16 * k, 16)]
                    plsc.store_scatter(bouts[par], [rows16[k], fvec], vals)
                return fvec + one

        in_start(lo, 0)
        in_start(lo + 1, 1)

        @pl.loop(0, cnt, step=2)
        def _pair(p):
            for par in range(2):
                c = lo + p + par

                @pl.when(p > 0)
                def _():
                    out_wait(par)

                in_wait(par)
                transpose(par)
                out_start(c, par)

                @pl.when(c + 2 < hi)
                def _():
                    in_start(c + 2, par)

        out_wait(0)
        out_wait(1)

        @pl.when(wid == NW - 1)
        def _tail():
            pltpu.sync_copy(tail_hbm, bin0)
            pltpu.sync_copy(bin0.at[pl.ds(0, TAIL), :],
                            tpad_hbm.at[pl.ds(TCOLS * 128, TAIL), :])

    return ka


_NB = 5
_PF = 3


def _build_gather(N, V, D, NC, NCH, CH):
    n_per_w = NCH * CH
    NB, PF = _NB, _PF
    assert NCH % NB == 0 and NCH >= 2 * NB

    mesh = plsc.VectorSubcoreMesh(core_axis_name="c", subcore_axis_name="s")

    @functools.partial(
        pl.kernel,
        mesh=mesh,
        out_type=jax.ShapeDtypeStruct((N, 2 * D), _F32),
        scratch_types=[
            pltpu.VMEM((NCH, CH), _I32),
            *[pltpu.VMEM((CH, 2 * D), _F32) for _ in range(NB)],
            *[pltpu.SemaphoreType.DMA for _ in range(2 * NB)],
        ],
        compiler_params=_PARAMS,
    )
    def kb(idx_hbm, tpad_hbm, out_hbm, idx_v, *rest):
        bufs = rest[:NB]
        gsems = rest[NB:2 * NB]
        ssems = rest[2 * NB:]
        wid = lax.axis_index("s") * NC + lax.axis_index("c")
        base = wid * n_per_w
        pltpu.sync_copy(idx_hbm.at[wid], idx_v)

        def gather_start(c, s):
            pltpu.async_copy(tpad_hbm.at[idx_v.at[c]], bufs[s], gsems[s])

        def gather_wait(s):
            pltpu.make_async_copy(
                tpad_hbm.at[idx_v.at[0]], bufs[s], gsems[s]).wait()

        def scatter_start(c, s):
            pltpu.async_copy(
                bufs[s], out_hbm.at[pl.ds(base + c * CH, CH)], ssems[s])

        def scatter_wait(s):
            pltpu.make_async_copy(
                bufs[s], out_hbm.at[pl.ds(base, CH)], ssems[s]).wait()

        for c in range(PF):
            gather_start(c, c)
        for i in range(NB - PF):
            gather_start(i + PF, i + PF)
            gather_wait(i)
            scatter_start(i, i)

        @pl.loop(NB - PF, NCH - PF, step=NB)
        def _wave(w):
            for b in range(NB):
                i = w + b
                s = (NB - PF + b) % NB
                sn = (NB - PF + b + PF) % NB
                scatter_wait(sn)
                gather_start(i + PF, sn)
                gather_wait(s)
                scatter_start(i, s)

        for i in range(NCH - PF, NCH):
            s = i % NB
            sn = (i + PF) % NB
            scatter_wait(sn)
            gather_wait(s)
            scatter_start(i, s)
        for i in range(NCH - (NB - PF), NCH):
            scatter_wait(i % NB)

    return kb


def kernel(x, table):
    B, H = x.shape
    V, D = table.shape
    N = B * H

    info = plsc.get_sparse_core_info()
    NC, NS = info.num_cores, info.num_subcores
    NW = NC * NS
    CH = 128
    NCH = N // (NW * CH)
    assert NW * NCH * CH == N

    TCOLS = V // 128
    tt = table.T
    tail128 = jnp.pad(
        lax.slice(table, (TCOLS * 128, 0), (V, D)), ((0, 0), (0, 2 * D - D)))

    t_pad = _build_transpose(V, D, NC, NW)(tt, tail128)

    idx = x.reshape(NW, NCH, CH)
    out128 = _build_gather(N, V, D, NC, NCH, CH)(idx, t_pad)
    return out128[:, :D].reshape(B, H, D)


# speedup vs baseline: 1.4826x; 1.4826x over previous
import functools

import jax
import jax.numpy as jnp
from jax import lax
from jax.experimental import pallas as pl
from jax.experimental.pallas import tpu as pltpu
from jax.experimental.pallas import tpu_sc as plsc

_NB = 10
_PF = 5


def _build_kernel(N, D, NC, NCH, CH):
    n_per_w = NCH * CH
    NB, PF = _NB, _PF
    assert NCH % NB == 0 and NCH >= 2 * NB

    mesh = plsc.VectorSubcoreMesh(core_axis_name="c", subcore_axis_name="s")

    @functools.partial(
        pl.kernel,
        mesh=mesh,
        out_type=jax.ShapeDtypeStruct((N, D), jnp.float32),
        scratch_types=[
            pltpu.VMEM((NCH, CH), jnp.int32),
            *[pltpu.VMEM((CH, D), jnp.float32) for _ in range(NB)],
            *[pltpu.SemaphoreType.DMA for _ in range(2 * NB)],
        ],
        compiler_params=pltpu.CompilerParams(use_tc_tiling_on_sc=False),
    )
    def k(idx_hbm, table_hbm, out_hbm, idx_v, *rest):
        bufs = rest[:NB]
        gsems = rest[NB:2 * NB]
        ssems = rest[2 * NB:]
        wid = lax.axis_index("s") * NC + lax.axis_index("c")
        base = wid * n_per_w
        pltpu.sync_copy(idx_hbm.at[wid], idx_v)

        def gather_start(c, s):
            pltpu.async_copy(table_hbm.at[idx_v.at[c]], bufs[s], gsems[s])

        def gather_wait(s):
            pltpu.make_async_copy(
                table_hbm.at[idx_v.at[0]], bufs[s], gsems[s]).wait()

        def scatter_start(c, s):
            pltpu.async_copy(
                bufs[s], out_hbm.at[pl.ds(base + c * CH, CH)], ssems[s])

        def scatter_wait(s):
            pltpu.make_async_copy(
                bufs[s], out_hbm.at[pl.ds(base, CH)], ssems[s]).wait()

        for c in range(PF):
            gather_start(c, c)
        for i in range(PF):
            gather_start(i + PF, i + PF)
            gather_wait(i)
            scatter_start(i, i)

        @pl.loop(PF, NCH - PF, step=NB)
        def _wave(w):
            for b in range(NB):
                i = w + b
                s = (PF + b) % NB
                sn = b
                scatter_wait(sn)
                gather_start(i + PF, sn)
                gather_wait(s)
                scatter_start(i, s)

        for i in range(NCH - PF, NCH):
            s = i % NB
            sn = (i + PF) % NB
            scatter_wait(sn)
            gather_wait(s)
            scatter_start(i, s)
        for i in range(NCH - PF, NCH):
            scatter_wait(i % NB)

    return k


def kernel(x, table):
    B, H = x.shape
    V, D = table.shape
    N = B * H

    info = plsc.get_sparse_core_info()
    NC, NS = info.num_cores, info.num_subcores
    NW = NC * NS
    CH = 128
    NCH = N // (NW * CH)
    assert NW * NCH * CH == N

    idx = x.reshape(NW, NCH, CH)
    out = _build_kernel(N, D, NC, NCH, CH)(idx, table)
    return out.reshape(B, H, D)
